# baseline (speedup 1.0000x reference)
import functools

import jax
import jax.numpy as jnp
from jax import lax
from jax.experimental import pallas as pl
from jax.experimental.pallas import tpu as pltpu
from jax.experimental.pallas import tpu_sc as plsc

_VOCAB = 100000
_D = 64
_NT = 5
_SEQ = 200
_NG = _SEQ - _NT
_L = 16
_BB = 128


def _build(B, NC, NS):
    NW = NC * NS
    assert B % _BB == 0 and B // _BB == NW
    mesh = plsc.VectorSubcoreMesh(core_axis_name="c", subcore_axis_name="s")

    @functools.partial(
        pl.kernel,
        out_type=jax.ShapeDtypeStruct((_SEQ, _D // 8, B // _BB, 8, _BB),
                                      jnp.float32),
        mesh=mesh,
        compiler_params=pltpu.CompilerParams(use_tc_tiling_on_sc=False,
                                             needs_layout_passes=False),
        scratch_types=[
            pltpu.VMEM((_SEQ, _BB), jnp.int32),
            pltpu.VMEM((_NT * _D, _BB), jnp.float32),
            [pltpu.VMEM((_BB, _D), jnp.float32)] * 2,
            [pltpu.VMEM((_D // 8, 8, _BB), jnp.float32)] * 2,
            pltpu.VMEM((_NT * _D, _L), jnp.float32),
            pltpu.VMEM((_NT * _D, _L), jnp.float32),
            [pltpu.SemaphoreType.DMA] * 2,
            [pltpu.SemaphoreType.DMA] * 2,
        ],
    )
    def k(tok_hbm, table_hbm, samp_hbm, var_hbm, avg_hbm, out_hbm,
          idxs_v, samps_v, grows, planes, var_v, avg_v, sgs, sws):
        wid = lax.axis_index("s") * NC + lax.axis_index("c")
        b0 = wid * _BB
        pltpu.sync_copy(var_hbm, var_v)
        pltpu.sync_copy(avg_hbm, avg_v)
        pltpu.sync_copy(tok_hbm.at[:, pl.ds(b0, _BB)], idxs_v)
        pltpu.sync_copy(samp_hbm.at[:, pl.ds(b0, _BB)], samps_v)

        iota = lax.iota(jnp.int32, _L)

        def gather_issue(t, s):
            pltpu.async_copy(table_hbm.at[idxs_v.at[t + _NT]], grows[s],
                             sgs[s])

        def gather_wait(s):
            pltpu.make_async_copy(table_hbm.at[pl.ds(0, _BB)], grows[s],
                                  sgs[s]).wait()

        def write_issue(t, s):
            pltpu.async_copy(planes[s], out_hbm.at[t, :, wid], sws[s])

        def write_wait(s):
            pltpu.make_async_copy(planes[s], out_hbm.at[0, :, 0],
                                  sws[s]).wait()

        def transpose(s):
            g, p = grows[s], planes[s]
            for d in range(_D):
                dsplat = jnp.full((_L,), d, jnp.int32)
                for b16 in range(_BB // _L):
                    bidx = iota + (b16 * _L)
                    p[d // 8, d % 8, pl.ds(b16 * _L, _L)] = (
                        plsc.load_gather(g, [bidx, dsplat]))

        gather_issue(0, 0)

        def body(tt, carry):
            t0 = 2 * tt
            for s, t in ((0, t0), (1, t0 + 1)):
                o = 1 - s
                if s == 0:
                    gather_issue(t + 1, 1)

                @pl.when(tt > 0)
                def _():
                    write_wait(s)

                gather_wait(s)
                transpose(s)
                write_issue(t, s)

                @pl.when(t + 2 < _NG)
                def _():
                    gather_issue(t + 2, s)
            return carry

        lax.fori_loop(0, _NG // 2, body, 0)
        write_wait(0)
        gather_wait(0)
        transpose(0)
        write_issue(_NG - 1, 0)
        write_wait(1)
        write_wait(0)

        def prefix(r, carry):
            p = planes[0]
            for d in range(_D):
                j = r * _D + d
                v = var_v[j]
                a = avg_v[j]
                for b16 in range(_BB // _L):
                    sl = pl.ds(b16 * _L, _L)
                    p[d // 8, d % 8, sl] = samps_v[j, sl] * v + a
            pltpu.sync_copy(p, out_hbm.at[_NG + r, :, wid])
            return carry

        lax.fori_loop(0, _NT, prefix, 0)

    return k


def kernel(tokens, table, avg, var):
    B = tokens.shape[0]
    sample = jax.random.normal(jax.random.key(1), (B, _NT * _D),
                               dtype=jnp.float32)
    var_b = jnp.broadcast_to(var.reshape(_NT * _D)[:, None], (_NT * _D, _L))
    avg_b = jnp.broadcast_to(avg.reshape(_NT * _D)[:, None], (_NT * _D, _L))
    info = plsc.get_sparse_core_info()
    k = _build(B, info.num_cores, info.num_subcores)
    o5 = k(tokens.T, table, sample.T, var_b, avg_b)
    return jnp.transpose(o5, (2, 4, 0, 1, 3)).reshape(B, _SEQ, _D)


# speedup vs baseline: 1.9810x; 1.9810x over previous
import functools

import jax
import jax.numpy as jnp
from jax import lax
from jax.experimental import pallas as pl
from jax.experimental.pallas import tpu as pltpu
from jax.experimental.pallas import tpu_sc as plsc

_VOCAB = 100000
_D = 64
_NT = 5
_SEQ = 200
_NG = _SEQ - _NT
_L = 16


def _build(B, NC, NS):
    NW = NC * NS
    bpw = B // NW
    mesh = plsc.VectorSubcoreMesh(core_axis_name="c", subcore_axis_name="s")

    @functools.partial(
        pl.kernel,
        out_type=jax.ShapeDtypeStruct((B, _SEQ, _D), jnp.float32),
        mesh=mesh,
        compiler_params=pltpu.CompilerParams(use_tc_tiling_on_sc=False),
        scratch_types=[
            pltpu.VMEM((bpw, _SEQ), jnp.int32),
            pltpu.VMEM((bpw, _NT * _D), jnp.float32),
            [pltpu.VMEM((_SEQ + _NT, _D), jnp.float32)] * 2,
            pltpu.VMEM((_NT, _D), jnp.float32),
            pltpu.VMEM((_NT, _D), jnp.float32),
            [pltpu.SemaphoreType.DMA] * 2,
        ],
    )
    def k(tok_hbm, table_hbm, samp_hbm, var_hbm, avg_hbm, out_hbm,
          idxs_v, samps_v, bufs, var_v, avg_v, sgs):
        wid = lax.axis_index("s") * NC + lax.axis_index("c")
        b0 = wid * bpw
        pltpu.sync_copy(var_hbm, var_v)
        pltpu.sync_copy(avg_hbm, avg_v)
        pltpu.sync_copy(tok_hbm.at[pl.ds(b0, bpw)], idxs_v)
        pltpu.sync_copy(samp_hbm.at[pl.ds(b0, bpw)], samps_v)

        def gather_issue(g, s):
            pltpu.async_copy(table_hbm.at[idxs_v.at[g, pl.ds(0, 128)]],
                             bufs[s].at[pl.ds(0, 128)], sgs[s])
            pltpu.async_copy(table_hbm.at[idxs_v.at[g, pl.ds(128, 72)]],
                             bufs[s].at[pl.ds(128, 72)], sgs[s])

        def finish(g, s):
            buf = bufs[s]
            for j in range(_NT * _D // _L):
                r, c = divmod(j, _D // _L)
                csl = pl.ds(c * _L, _L)
                buf[_SEQ + r, csl] = (
                    samps_v[g, pl.ds(j * _L, _L)] * var_v[r, csl]
                    + avg_v[r, csl])
            pltpu.make_async_copy(table_hbm.at[pl.ds(0, 128)],
                                  buf.at[pl.ds(0, 128)], sgs[s]).wait()
            pltpu.make_async_copy(table_hbm.at[pl.ds(0, 72)],
                                  buf.at[pl.ds(128, 72)], sgs[s]).wait()
            pltpu.sync_copy(buf.at[pl.ds(_NT, _SEQ)], out_hbm.at[b0 + g])

        gather_issue(0, 0)

        def body(tt, carry):
            g0 = 2 * tt
            gather_issue(g0 + 1, 1)
            finish(g0, 0)

            @pl.when(tt + 1 < bpw // 2)
            def _():
                gather_issue(g0 + 2, 0)

            finish(g0 + 1, 1)
            return carry

        lax.fori_loop(0, bpw // 2, body, 0)

    return k


def kernel(tokens, table, avg, var):
    B = tokens.shape[0]
    sample = jax.random.normal(jax.random.key(1), (B, _NT * _D),
                               dtype=jnp.float32)
    info = plsc.get_sparse_core_info()
    k = _build(B, info.num_cores, info.num_subcores)
    return k(tokens, table, sample, var, avg)
